# Initial kernel scaffold; baseline (speedup 1.0000x reference)
import jax
import jax.numpy as jnp
from jax.experimental import pallas as pl


def kernel(x, Wr, Wg, Wu, Wd):
    raise NotImplementedError("write your pallas kernel here")


# speedup vs baseline: 1.4001x; 1.4001x over previous
import jax
import jax.numpy as jnp
from jax.experimental import pallas as pl
from jax.experimental.pallas import tpu as pltpu


def _moe_body(x_ref, wr_ref, wg_ref, wu_ref, wd_ref, out_ref, w_scr):
    e = pl.program_id(0)
    n_experts = pl.num_programs(0)

    @pl.when(e == 0)
    def _router():
        x = x_ref[...]
        logits = jax.lax.dot_general(
            x, wr_ref[...], (((1,), (1,)), ((), ())),
            preferred_element_type=jnp.float32)
        m = jnp.max(logits, axis=-1, keepdims=True)
        ex = jnp.exp(logits - m)
        scores = ex / jnp.sum(ex, axis=-1, keepdims=True)
        idx = jax.lax.broadcasted_iota(jnp.int32, scores.shape, 1)
        m1 = jnp.max(scores, axis=-1, keepdims=True)
        i1 = jnp.min(jnp.where(scores == m1, idx, n_experts), axis=-1,
                     keepdims=True)
        oh1 = idx == i1
        s2 = jnp.where(oh1, -1.0, scores)
        m2 = jnp.max(s2, axis=-1, keepdims=True)
        i2 = jnp.min(jnp.where(s2 == m2, idx, n_experts), axis=-1,
                     keepdims=True)
        oh2 = idx == i2
        w_scr[...] = jnp.where(oh1, m1, 0.0) + jnp.where(oh2, m2, 0.0)
        out_ref[...] = jnp.zeros_like(out_ref)

    x = x_ref[...]
    wg = wg_ref[0]
    wu = wu_ref[0]
    wd = wd_ref[0]
    h1 = jnp.dot(x, wg, preferred_element_type=jnp.float32)
    h2 = jnp.dot(x, wu, preferred_element_type=jnp.float32)
    g = h1 * (1.0 / (1.0 + jnp.exp(-h1)))
    y = jnp.dot(g * h2, wd, preferred_element_type=jnp.float32)
    w = w_scr[...]
    lane = jax.lax.broadcasted_iota(jnp.int32, w.shape, 1)
    wcol = jnp.sum(jnp.where(lane == e, w, 0.0), axis=1, keepdims=True)
    out_ref[...] += y * wcol


def kernel(x, Wr, Wg, Wu, Wd):
    b, s, h = x.shape
    e, _, f = Wg.shape
    n = b * s
    xf = x.reshape(n, h)
    out = pl.pallas_call(
        _moe_body,
        grid=(e,),
        in_specs=[
            pl.BlockSpec((n, h), lambda i: (0, 0)),
            pl.BlockSpec((e, h), lambda i: (0, 0)),
            pl.BlockSpec((1, h, f), lambda i: (i, 0, 0)),
            pl.BlockSpec((1, h, f), lambda i: (i, 0, 0)),
            pl.BlockSpec((1, f, h), lambda i: (i, 0, 0)),
        ],
        out_specs=pl.BlockSpec((n, h), lambda i: (0, 0)),
        out_shape=jax.ShapeDtypeStruct((n, h), jnp.float32),
        scratch_shapes=[pltpu.VMEM((n, e), jnp.float32)],
        compiler_params=pltpu.CompilerParams(
            dimension_semantics=("arbitrary",)),
    )(xf, Wr, Wg, Wu, Wd)
    return out.reshape(b, s, h)


# speedup vs baseline: 1.4931x; 1.0664x over previous
import jax
import jax.numpy as jnp
from jax.experimental import pallas as pl
from jax.experimental.pallas import tpu as pltpu

_EPP = 2


def _moe_body(x_ref, wr_ref, wg_ref, wu_ref, wd_ref, out_ref, w_scr):
    step = pl.program_id(0)
    n_experts = pl.num_programs(0) * _EPP

    @pl.when(step == 0)
    def _router():
        x = x_ref[...]
        logits = jax.lax.dot_general(
            x, wr_ref[...], (((1,), (1,)), ((), ())),
            preferred_element_type=jnp.float32)
        m = jnp.max(logits, axis=-1, keepdims=True)
        ex = jnp.exp(logits - m)
        scores = ex / jnp.sum(ex, axis=-1, keepdims=True)
        idx = jax.lax.broadcasted_iota(jnp.int32, scores.shape, 1)
        m1 = jnp.max(scores, axis=-1, keepdims=True)
        i1 = jnp.min(jnp.where(scores == m1, idx, n_experts), axis=-1,
                     keepdims=True)
        oh1 = idx == i1
        s2 = jnp.where(oh1, -1.0, scores)
        m2 = jnp.max(s2, axis=-1, keepdims=True)
        i2 = jnp.min(jnp.where(s2 == m2, idx, n_experts), axis=-1,
                     keepdims=True)
        oh2 = idx == i2
        w_scr[...] = jnp.where(oh1, m1, 0.0) + jnp.where(oh2, m2, 0.0)
        out_ref[...] = jnp.zeros_like(out_ref)

    x = x_ref[...]
    w = w_scr[...]
    lane = jax.lax.broadcasted_iota(jnp.int32, w.shape, 1)
    acc = jnp.zeros_like(out_ref)
    for j in range(_EPP):
        eid = step * _EPP + j
        h1 = jnp.dot(x, wg_ref[j], preferred_element_type=jnp.float32)
        h2 = jnp.dot(x, wu_ref[j], preferred_element_type=jnp.float32)
        g = h1 * (1.0 / (1.0 + jnp.exp(-h1)))
        y = jnp.dot(g * h2, wd_ref[j], preferred_element_type=jnp.float32)
        wcol = jnp.sum(jnp.where(lane == eid, w, 0.0), axis=1, keepdims=True)
        acc = acc + y * wcol
    out_ref[...] += acc


def kernel(x, Wr, Wg, Wu, Wd):
    b, s, h = x.shape
    e, _, f = Wg.shape
    n = b * s
    xf = x.reshape(n, h)
    out = pl.pallas_call(
        _moe_body,
        grid=(e // _EPP,),
        in_specs=[
            pl.BlockSpec((n, h), lambda i: (0, 0)),
            pl.BlockSpec((e, h), lambda i: (0, 0)),
            pl.BlockSpec((_EPP, h, f), lambda i: (i, 0, 0)),
            pl.BlockSpec((_EPP, h, f), lambda i: (i, 0, 0)),
            pl.BlockSpec((_EPP, f, h), lambda i: (i, 0, 0)),
        ],
        out_specs=pl.BlockSpec((n, h), lambda i: (0, 0)),
        out_shape=jax.ShapeDtypeStruct((n, h), jnp.float32),
        scratch_shapes=[pltpu.VMEM((n, e), jnp.float32)],
        compiler_params=pltpu.CompilerParams(
            dimension_semantics=("arbitrary",)),
    )(xf, Wr, Wg, Wu, Wd)
    return out.reshape(b, s, h)
